# Initial kernel scaffold; baseline (speedup 1.0000x reference)
import jax
import jax.numpy as jnp
from jax.experimental import pallas as pl


def kernel(x, edge_index, edge_attr, expander_edge_index, expander_node_mask, params):
    raise NotImplementedError("write your pallas kernel here")


# speedup vs baseline: 4.9830x; 4.9830x over previous
import functools

import jax
import jax.numpy as jnp
from jax import lax
from jax.experimental import pallas as pl
from jax.experimental.pallas import tpu as pltpu
from jax.experimental.pallas import tpu_sc as plsc

N = 10000
E = 320000
D = 128
L = 4
V = 1001
H = 256

NC = 2
NS = 16
NW = NC * NS

EPW = E // NW
CH = 125
NCH = EPW // CH
IG = 8
NIG = NCH // IG

NPAD = 10240
RPT = NPAD // NS
ZCH = 80
NZCH = RPT // ZCH

NP = 10240
RPW = NP // NW
ECH = 80
NECH = RPW // ECH

_mesh = plsc.VectorSubcoreMesh(
    core_axis_name="c", subcore_axis_name="s", num_cores=NC, num_subcores=NS)

_f32 = jnp.float32


def _zero_rows_buf(buf, nrows):
  z = jnp.zeros((16,), _f32)

  def body(r, _):
    for j in range(D // 16):
      buf[r, pl.ds(j * 16, 16)] = z
    return 0

  lax.fori_loop(0, nrows, body, 0)


def _make_seg(with_ea):
  scratch = [
      pltpu.VMEM((IG, CH), jnp.int32),
      pltpu.VMEM((IG, CH), jnp.int32),
      pltpu.VMEM((CH, D), _f32),
      pltpu.VMEM_SHARED((NPAD, D), _f32),
      pltpu.SemaphoreType.DMA,
  ]
  if with_ea:
    scratch.insert(3, pltpu.VMEM((CH, D), _f32))

  def body(*refs):
    if with_ea:
      (h_hbm, src_hbm, dst_hbm, ea_hbm, out_hbm,
       sidx, didx, rows, earows, aggr, gsem) = refs
    else:
      (h_hbm, src_hbm, dst_hbm, out_hbm,
       sidx, didx, rows, aggr, gsem) = refs
    c = lax.axis_index("c")
    s = lax.axis_index("s")
    w = s * NC + c

    _zero_rows_buf(rows, ZCH)
    for q in range(NZCH):
      pltpu.sync_copy(rows.at[pl.ds(0, ZCH)],
                      aggr.at[pl.ds(s * RPT + q * ZCH, ZCH)])
    plsc.subcore_barrier()

    def group(grp, _):
      pltpu.sync_copy(src_hbm.at[w, pl.ds(grp * IG, IG)], sidx)
      pltpu.sync_copy(dst_hbm.at[w, pl.ds(grp * IG, IG)], didx)

      def chunk(j, _):
        pltpu.async_copy(h_hbm.at[sidx.at[j]], rows, gsem).wait()
        if with_ea:
          pltpu.sync_copy(ea_hbm.at[w, grp * IG + j], earows)

          def relu_row(r, _):
            for k in range(D // 16):
              v = rows[r, pl.ds(k * 16, 16)] + earows[r, pl.ds(k * 16, 16)]
              rows[r, pl.ds(k * 16, 16)] = jnp.maximum(v, 0.0)
            return 0

          lax.fori_loop(0, CH, relu_row, 0)
        pltpu.sync_copy(rows, aggr.at[didx.at[j]], add=True)
        return 0

      lax.fori_loop(0, IG, chunk, 0)
      return 0

    lax.fori_loop(0, NIG, group, 0)
    plsc.subcore_barrier()

    pltpu.sync_copy(aggr.at[pl.ds(s * RPT, RPT)],
                    out_hbm.at[c, pl.ds(s * RPT, RPT)])

  out_type = jax.ShapeDtypeStruct((NC, NPAD, D), _f32)
  return pl.kernel(body, out_type=out_type, mesh=_mesh, scratch_types=scratch,
                   name="seg_sum_ea" if with_ea else "seg_sum")


_seg_ea = _make_seg(True)
_seg = _make_seg(False)


def _embed_body(keys_hbm, vals_hbm, i0_hbm, i1_hbm, out_hbm,
                i0v, i1v, ra, rb, sem0, sem1):
  c = lax.axis_index("c")
  s = lax.axis_index("s")
  w = s * NC + c
  pltpu.sync_copy(i0_hbm.at[w], i0v)
  pltpu.sync_copy(i1_hbm.at[w], i1v)

  def chunk(g, _):
    pltpu.async_copy(keys_hbm.at[i0v.at[g]], ra, sem0).wait()
    pltpu.async_copy(vals_hbm.at[i1v.at[g]], rb, sem1).wait()

    def add_row(r, _):
      for j in range(D // 16):
        ra[r, pl.ds(j * 16, 16)] = (
            ra[r, pl.ds(j * 16, 16)] + rb[r, pl.ds(j * 16, 16)])
      return 0

    lax.fori_loop(0, ECH, add_row, 0)
    pltpu.sync_copy(ra, out_hbm.at[pl.ds(w * RPW + g * ECH, ECH)])
    return 0

  lax.fori_loop(0, NECH, chunk, 0)


_embed = pl.kernel(
    _embed_body,
    out_type=jax.ShapeDtypeStruct((NP, D), _f32),
    mesh=_mesh,
    scratch_types=[
        pltpu.VMEM((NECH, ECH), jnp.int32),
        pltpu.VMEM((NECH, ECH), jnp.int32),
        pltpu.VMEM((ECH, D), _f32),
        pltpu.VMEM((ECH, D), _f32),
        pltpu.SemaphoreType.DMA,
        pltpu.SemaphoreType.DMA,
    ],
    name="embed")


def _make_tc_prop(update_original, with_ln):
  def body(*refs):
    if with_ln:
      (h_ref, agg_ref, w1_ref, b1_ref, w2_ref, b2_ref, ep_ref,
       g_ref, b_ref, mk_ref, lg_ref, lb_ref, out_ref) = refs
    else:
      (h_ref, agg_ref, w1_ref, b1_ref, w2_ref, b2_ref, ep_ref,
       g_ref, b_ref, mk_ref, out_ref) = refs
    h = h_ref[...]
    agg = agg_ref[0, :N, :] + agg_ref[1, :N, :]
    z = ep_ref[...] * h + agg
    u = jnp.maximum(
        jnp.dot(z, w1_ref[...], preferred_element_type=_f32) + b1_ref[...],
        0.0)
    z2 = jnp.dot(u, w2_ref[...], preferred_element_type=_f32) + b2_ref[...]
    m = mk_ref[...]
    if update_original:
      hc = m * z2 + (1.0 - m) * h
    else:
      hc = m * h + (1.0 - m) * z2
    mu = jnp.mean(hc, axis=0, keepdims=True)
    var = jnp.mean((hc - mu) * (hc - mu), axis=0, keepdims=True)
    hb = g_ref[...] * (hc - mu) * lax.rsqrt(var + 1e-5) + b_ref[...]
    hn = jnp.maximum(hb, 0.0) + h
    if with_ln:
      mu2 = jnp.mean(hn, axis=1, keepdims=True)
      v2 = jnp.mean((hn - mu2) * (hn - mu2), axis=1, keepdims=True)
      hn = lg_ref[...] * (hn - mu2) * lax.rsqrt(v2 + 1e-5) + lb_ref[...]
    out_ref[...] = hn

  return pl.pallas_call(
      body, out_shape=jax.ShapeDtypeStruct((N, D), _f32))


_tc_orig = _make_tc_prop(True, False)
_tc_exp = _make_tc_prop(False, False)
_tc_orig_ln = _make_tc_prop(True, True)


def kernel(x, edge_index, edge_attr, expander_edge_index, expander_node_mask,
           params):
  p = params
  maskf = expander_node_mask.astype(_f32)[:, None]

  keys_z = jnp.concatenate([p['keys_table'], jnp.zeros((1, D), _f32)], axis=0)
  vals_z = jnp.concatenate([p['values_table'], jnp.zeros((1, D), _f32)],
                           axis=0)
  zpad = jnp.full((NP - N,), V, jnp.int32)
  i0 = jnp.concatenate(
      [jnp.where(expander_node_mask > 0, x[:, 0], V), zpad]).reshape(
          NW, NECH, ECH)
  i1 = jnp.concatenate(
      [jnp.where(expander_node_mask > 0, x[:, 1], V), zpad]).reshape(
          NW, NECH, ECH)
  h = _embed(keys_z, vals_z, i0, i1)[:N]

  src_c = edge_index[0].reshape(NW, NCH, CH)
  dst_c = edge_index[1].reshape(NW, NCH, CH)
  ea_r = edge_attr.reshape(NW, NCH, CH, D)
  src_x = expander_edge_index[0].reshape(NW, NCH, CH)
  dst_x = expander_edge_index[1].reshape(NW, NCH, CH)

  def b2d(v):
    return v.reshape(1, -1)

  one = jnp.ones((1, 1), _f32)

  for l in range(L):
    agg = _seg_ea(h, src_c, dst_c, ea_r)
    h = _tc_orig(h, agg, p['conv_W1'][l], b2d(p['conv_b1'][l]),
                 p['conv_W2'][l], b2d(p['conv_b2'][l]),
                 one + p['conv_eps'][l], b2d(p['bn_gamma'][l]),
                 b2d(p['bn_beta'][l]), maskf)
    agg = _seg(h, src_x, dst_x)
    h = _tc_exp(h, agg, p['left_W1'][l], b2d(p['left_b1'][l]),
                p['left_W2'][l], b2d(p['left_b2'][l]),
                one + p['left_eps'][l], b2d(p['left_bn_gamma'][l]),
                b2d(p['left_bn_beta'][l]), maskf)
    agg = _seg(h, dst_x, src_x)
    h = _tc_orig_ln(h, agg, p['right_W1'][l], b2d(p['right_b1'][l]),
                    p['right_W2'][l], b2d(p['right_b2'][l]),
                    one + p['right_eps'][l], b2d(p['right_bn_gamma'][l]),
                    b2d(p['right_bn_beta'][l]), maskf,
                    b2d(p['ln_gamma'][l]), b2d(p['ln_beta'][l]))
  return h
